# Initial kernel scaffold; baseline (speedup 1.0000x reference)
import jax
import jax.numpy as jnp
from jax.experimental import pallas as pl


def kernel(theta, phi, dist, omega, mask, idx_theta, idx_phi, idx_dist, idx_omega):
    raise NotImplementedError("write your pallas kernel here")


# speedup vs baseline: 2.2895x; 2.2895x over previous
import functools

import jax
import jax.numpy as jnp
from jax.experimental import pallas as pl
from jax.experimental.pallas import tpu as pltpu

L = 512
NB_THETA, NB_PHI, NB_DIST, NB_OMEGA = 25, 13, 37, 25
BR = 16


def _body(theta_ref, phi_ref, dist_ref, omega_ref, mask_ref,
          it_ref, ip_ref, id_ref, io_ref, out_ref):
    m = mask_ref[...]
    acc = jnp.zeros((BR, L), jnp.float32)
    for ref, iref, nb in ((theta_ref, it_ref, NB_THETA),
                          (phi_ref, ip_ref, NB_PHI),
                          (dist_ref, id_ref, NB_DIST),
                          (omega_ref, io_ref, NB_OMEGA)):
        idx = iref[0]
        sel = ref[0, 0]
        for b in range(1, nb):
            sel = jnp.where(idx == b, ref[0, b], sel)
        acc = acc + jnp.log(sel)
    part = jnp.sum(acc * m)

    @pl.when(pl.program_id(0) == 0)
    def _():
        out_ref[0, 0] = 0.0

    out_ref[0, 0] += part


@jax.jit
def kernel(theta, phi, dist, omega, mask, idx_theta, idx_phi, idx_dist, idx_omega):
    grid = (L // BR,)

    def dist_spec(nb):
        return pl.BlockSpec((1, nb, BR, L), lambda i: (0, 0, i, 0))

    idx_spec = pl.BlockSpec((1, BR, L), lambda i: (0, i, 0))

    total = pl.pallas_call(
        _body,
        grid=grid,
        in_specs=[
            dist_spec(NB_THETA),
            dist_spec(NB_PHI),
            dist_spec(NB_DIST),
            dist_spec(NB_OMEGA),
            pl.BlockSpec((BR, L), lambda i: (i, 0)),
            idx_spec, idx_spec, idx_spec, idx_spec,
        ],
        out_specs=pl.BlockSpec(memory_space=pltpu.SMEM),
        out_shape=jax.ShapeDtypeStruct((1, 1), jnp.float32),
    )(theta, phi, dist, omega, mask,
      idx_theta, idx_phi, idx_dist, idx_omega)
    return -total[0, 0] / jnp.float32(L * L)
